# baseline (speedup 1.0000x reference)
import functools

import jax
import jax.numpy as jnp
from jax import lax
from jax.experimental import pallas as pl
from jax.experimental.pallas import tpu as pltpu
from jax.experimental.pallas import tpu_sc as plsc

VOCAB = 1_000_000
DIM = 32
NUM_IDX = 16384 * 50
GROUP = 128
NUM_GROUPS = NUM_IDX // GROUP
NUM_WORKERS = 32
GPW = NUM_GROUPS // NUM_WORKERS
K = 10
T = GPW // K
THALF = T // 2
BATCH_ROWS = K * GROUP


def _body(x_hbm, emb_hbm, out_hbm, idx_v, rows_v, gsem_a, gsem_b, osem_a, osem_b):
    c = lax.axis_index("c")
    s = lax.axis_index("s")
    wid = s * 2 + c
    gbase = wid * GPW
    pltpu.sync_copy(x_hbm.at[pl.ds(gbase * GROUP, GPW * GROUP)], idx_v)

    def fire(batch, half, sem):
        descs = []
        for b in range(K):
            g = batch * K + b
            descs.append(
                pltpu.async_copy(
                    emb_hbm.at[idx_v.at[pl.ds(g * GROUP, GROUP)]],
                    rows_v.at[pl.ds((half * K + b) * GROUP, GROUP)],
                    sem,
                )
            )
        return descs

    def out_copy(batch, half, sem):
        row0 = (gbase + batch * K) * GROUP
        return pltpu.async_copy(
            rows_v.at[pl.ds(half * BATCH_ROWS, BATCH_ROWS)],
            out_hbm.at[pl.ds(row0, BATCH_ROWS)],
            sem,
        )

    def outer(t, carry):
        ga = fire(2 * t, 0, gsem_a)
        gb = fire(2 * t + 1, 1, gsem_b)
        for d in ga:
            d.wait()
        oa = out_copy(2 * t, 0, osem_a)
        for d in gb:
            d.wait()
        ob = out_copy(2 * t + 1, 1, osem_b)
        oa.wait()
        ob.wait()
        return carry

    lax.fori_loop(0, THALF, outer, 0)


@jax.jit
def _gather(xflat, embeddings):
    f = functools.partial(
        pl.kernel,
        out_type=jax.ShapeDtypeStruct((NUM_IDX, DIM), jnp.float32),
        mesh=plsc.VectorSubcoreMesh(core_axis_name="c", subcore_axis_name="s"),
        scratch_types=[
            pltpu.VMEM((GPW * GROUP,), jnp.int32),
            pltpu.VMEM((2 * BATCH_ROWS, DIM), jnp.float32),
            pltpu.SemaphoreType.DMA,
            pltpu.SemaphoreType.DMA,
            pltpu.SemaphoreType.DMA,
            pltpu.SemaphoreType.DMA,
        ],
        compiler_params=pltpu.CompilerParams(use_tc_tiling_on_sc=False),
    )(_body)
    return f(xflat, embeddings)


def kernel(x, embeddings):
    xflat = x.reshape(NUM_IDX)
    out = _gather(xflat, embeddings)
    return out.reshape(x.shape[0], x.shape[1], DIM)


# speedup vs baseline: 1.6179x; 1.6179x over previous
import functools

import jax
import jax.numpy as jnp
from jax import lax
from jax.experimental import pallas as pl
from jax.experimental.pallas import tpu as pltpu
from jax.experimental.pallas import tpu_sc as plsc

VOCAB = 1_000_000
DIM = 32
XROWS = 16384
XCOLS = 50
NUM_IDX = XROWS * XCOLS
NUM_WORKERS = 32
IPW = NUM_IDX // NUM_WORKERS
XR_PER_WORKER = XROWS // NUM_WORKERS
XR_PER_BATCH = 16
BATCH_ROWS = XR_PER_BATCH * XCOLS
T = XR_PER_WORKER // XR_PER_BATCH
THALF = T // 2
GATHER_SIZES = (128, 128, 128, 128, 128, 128, 32)


def _body(x_hbm, emb_hbm, out_hbm, idx_v, rows_v, gsem_a, gsem_b, osem_a, osem_b):
    c = lax.axis_index("c")
    s = lax.axis_index("s")
    wid = s * 2 + c
    ibase = wid * IPW
    xrbase = wid * XR_PER_WORKER
    pltpu.sync_copy(x_hbm.at[pl.ds(ibase, IPW)], idx_v)

    def fire(batch, half, sem):
        descs = []
        off = 0
        for gsize in GATHER_SIZES:
            descs.append(
                pltpu.async_copy(
                    emb_hbm.at[idx_v.at[pl.ds(batch * BATCH_ROWS + off, gsize)]],
                    rows_v.at[pl.ds(half * BATCH_ROWS + off, gsize)],
                    sem,
                )
            )
            off += gsize
        return descs

    def out_copy(batch, half, sem):
        descs = []
        xr0 = xrbase + batch * XR_PER_BATCH
        for j in range(XR_PER_BATCH):
            descs.append(
                pltpu.async_copy(
                    rows_v.at[pl.ds(half * BATCH_ROWS + j * XCOLS, XCOLS)],
                    out_hbm.at[xr0 + j],
                    sem,
                )
            )
        return descs

    def outer(t, carry):
        ga = fire(2 * t, 0, gsem_a)
        gb = fire(2 * t + 1, 1, gsem_b)
        for d in ga:
            d.wait()
        oa = out_copy(2 * t, 0, osem_a)
        for d in gb:
            d.wait()
        ob = out_copy(2 * t + 1, 1, osem_b)
        for d in oa:
            d.wait()
        for d in ob:
            d.wait()
        return carry

    lax.fori_loop(0, THALF, outer, 0)


@jax.jit
def _gather(xflat, embeddings):
    f = functools.partial(
        pl.kernel,
        out_type=jax.ShapeDtypeStruct((XROWS, XCOLS, DIM), jnp.float32),
        mesh=plsc.VectorSubcoreMesh(core_axis_name="c", subcore_axis_name="s"),
        scratch_types=[
            pltpu.VMEM((IPW,), jnp.int32),
            pltpu.VMEM((2 * BATCH_ROWS, DIM), jnp.float32),
            pltpu.SemaphoreType.DMA,
            pltpu.SemaphoreType.DMA,
            pltpu.SemaphoreType.DMA,
            pltpu.SemaphoreType.DMA,
        ],
        compiler_params=pltpu.CompilerParams(use_tc_tiling_on_sc=False),
    )(_body)
    return f(xflat, embeddings)


def kernel(x, embeddings):
    xflat = x.reshape(NUM_IDX)
    return _gather(xflat, embeddings)
